# baseline (speedup 1.0000x reference)
import functools

import jax
import jax.numpy as jnp
from jax import lax
from jax.experimental import pallas as pl
from jax.experimental.pallas import tpu as pltpu
from jax.experimental.pallas import tpu_sc as plsc

N = 10000
E = 320000
DF = 128
DE = 16
DH = 64
G = 16
ITERS = 8

EB = 3200
NB = E // EB
NBLK = 1000
CH = 128
NCHUNK = E // CH
NW = 32

_AtB = (((0,), (0,)), ((), ()))


def _node_proj_body(x_ref, batchc_ref, wab_ref, ya_ref, yb_ref, cnt_ref,
                    brep_ref):
    b = pl.program_id(0)
    t = jnp.dot(x_ref[...], wab_ref[...], preferred_element_type=jnp.float32)
    ya_ref[...] = t[:, 0:80]
    yb_ref[...] = t[:, 80:160]
    bc = batchc_ref[0, :, :]
    brep_ref[...] = jnp.broadcast_to(bc, (NBLK, 16))
    oh = (bc == lax.broadcasted_iota(jnp.int32, (NBLK, G), 1)).astype(jnp.float32)
    ones = jnp.ones((NBLK, 1), dtype=jnp.float32)
    acc = lax.dot_general(oh, ones, _AtB, preferred_element_type=jnp.float32)

    @pl.when(b == 0)
    def _():
        cnt_ref[...] = jnp.zeros_like(cnt_ref)

    cnt_ref[...] += acc


def _node_proj(x, batchc, wab):
    nb = N // NBLK
    return pl.pallas_call(
        _node_proj_body,
        grid=(nb,),
        in_specs=[
            pl.BlockSpec((NBLK, DF), lambda b: (b, 0)),
            pl.BlockSpec((1, NBLK, 1), lambda b: (b, 0, 0)),
            pl.BlockSpec((DF, 160), lambda b: (0, 0)),
        ],
        out_specs=[
            pl.BlockSpec((NBLK, 80), lambda b: (b, 0)),
            pl.BlockSpec((NBLK, 80), lambda b: (b, 0)),
            pl.BlockSpec((G, 1), lambda b: (0, 0)),
            pl.BlockSpec((NBLK, 16), lambda b: (b, 0)),
        ],
        out_shape=[
            jax.ShapeDtypeStruct((N, 80), jnp.float32),
            jax.ShapeDtypeStruct((N, 80), jnp.float32),
            jax.ShapeDtypeStruct((G, 1), jnp.float32),
            jax.ShapeDtypeStruct((N, 16), jnp.int32),
        ],
        compiler_params=pltpu.CompilerParams(
            dimension_semantics=("arbitrary",)),
    )(x, batchc, wab)


def _gather_body(ya_hbm, yb_hbm, brep_hbm, rol_hbm, col_hbm,
                 yar_out, ybc_out, seg16_out,
                 idx_a, idx_b, rows_a, rows_b, rows_s, sem):
    cid = lax.axis_index("c")
    sid = lax.axis_index("s")
    wid = sid * 2 + cid
    nmine = (NCHUNK - wid + NW - 1) // NW

    def chunk(k, carry):
        c = wid + k * NW
        base = c * CH
        pltpu.sync_copy(rol_hbm.at[pl.ds(base, CH)], idx_a)
        pltpu.sync_copy(col_hbm.at[pl.ds(base, CH)], idx_b)
        cp_a = pltpu.async_copy(ya_hbm.at[idx_a], rows_a, sem)
        cp_b = pltpu.async_copy(yb_hbm.at[idx_b], rows_b, sem)
        cp_s = pltpu.async_copy(brep_hbm.at[idx_a], rows_s, sem)
        cp_a.wait()
        cp_b.wait()
        cp_s.wait()
        pltpu.sync_copy(rows_a, yar_out.at[pl.ds(base, CH)])
        pltpu.sync_copy(rows_b, ybc_out.at[pl.ds(base, CH)])
        pltpu.sync_copy(rows_s, seg16_out.at[pl.ds(base, CH)])
        return carry

    lax.fori_loop(0, nmine, chunk, 0)


def _edge_gather(ya, yb, brep, rol, col):
    mesh = plsc.VectorSubcoreMesh(core_axis_name="c", subcore_axis_name="s")
    fn = pl.kernel(
        _gather_body,
        out_type=[
            jax.ShapeDtypeStruct((E, 80), jnp.float32),
            jax.ShapeDtypeStruct((E, 80), jnp.float32),
            jax.ShapeDtypeStruct((E, 16), jnp.int32),
        ],
        mesh=mesh,
        scratch_types=[
            pltpu.VMEM((CH,), jnp.int32),
            pltpu.VMEM((CH,), jnp.int32),
            pltpu.VMEM((CH, 80), jnp.float32),
            pltpu.VMEM((CH, 80), jnp.float32),
            pltpu.VMEM((CH, 16), jnp.int32),
            pltpu.SemaphoreType.DMA,
        ],
        compiler_params=pltpu.CompilerParams(use_tc_tiling_on_sc=False),
    )
    return fn(ya, yb, brep, rol, col)


def _combine_body(yar_ref, ybc_ref, ea_ref, seg16_ref, wep_ref, bep_ref,
                  w3_ref, wsm_ref, scal_ref,
                  bh_ref, attc_ref, segc_ref, st_ref, pooled_s, cnt_s):
    b = pl.program_id(0)

    @pl.when(b == 0)
    def _():
        pooled_s[...] = jnp.zeros_like(pooled_s)
        cnt_s[...] = jnp.zeros_like(cnt_s)

    t = jnp.dot(ea_ref[...], wep_ref[...],
                preferred_element_type=jnp.float32) + bep_ref[...]
    base = yar_ref[:, 0:64] + ybc_ref[:, 0:64] + t[:, 0:64]
    hid0 = t[:, 64:128]
    bh_ref[...] = jnp.concatenate([base, hid0], axis=1)
    attc_ref[0, :, :] = (yar_ref[:, 64:65] + ybc_ref[:, 64:65]
                         + t[:, 128:129])
    segc = seg16_ref[:, 0:1]
    segc_ref[0, :, :] = segc

    oh = (segc == lax.broadcasted_iota(jnp.int32, (EB, G), 1)
          ).astype(jnp.float32)
    cur0 = jnp.tanh(base + jnp.dot(hid0, w3_ref[...],
                                   preferred_element_type=jnp.float32))
    pooled_s[...] += lax.dot_general(oh, cur0, _AtB,
                                     preferred_element_type=jnp.float32)
    ones = jnp.ones((EB, 1), dtype=jnp.float32)
    cnt_s[...] += lax.dot_general(oh, ones, _AtB,
                                  preferred_element_type=jnp.float32)

    @pl.when(b == NB - 1)
    def _():
        cnt = cnt_s[...]
        pooled_avg = pooled_s[...] / jnp.maximum(cnt, 1.0)
        wc = wsm_ref[:, 0:1]
        logits = jnp.dot(pooled_avg, wc, preferred_element_type=jnp.float32)
        conf0 = jax.nn.sigmoid(logits + scal_ref[0, 0])
        cidx = lax.broadcasted_iota(jnp.int32, (G, 128), 1)
        st_ref[...] = jnp.where(cidx == 0, conf0,
                                jnp.where(cidx == 1, 1.0,
                                          jnp.where(cidx == 2, cnt, 0.0)))


def _edge_combine(yar, ybc, ea, seg16, wep, bep, w3, wsm, scal):
    return pl.pallas_call(
        _combine_body,
        grid=(NB,),
        in_specs=[
            pl.BlockSpec((EB, 80), lambda b: (b, 0)),
            pl.BlockSpec((EB, 80), lambda b: (b, 0)),
            pl.BlockSpec((EB, DE), lambda b: (b, 0)),
            pl.BlockSpec((EB, 16), lambda b: (b, 0)),
            pl.BlockSpec((DE, 192), lambda b: (0, 0)),
            pl.BlockSpec((1, 192), lambda b: (0, 0)),
            pl.BlockSpec((DH, DH), lambda b: (0, 0)),
            pl.BlockSpec((DH, 128), lambda b: (0, 0)),
            pl.BlockSpec(memory_space=pltpu.SMEM),
        ],
        out_specs=[
            pl.BlockSpec((EB, 128), lambda b: (b, 0)),
            pl.BlockSpec((1, EB, 1), lambda b: (b, 0, 0)),
            pl.BlockSpec((1, EB, 1), lambda b: (b, 0, 0)),
            pl.BlockSpec((G, 128), lambda b: (0, 0)),
        ],
        out_shape=[
            jax.ShapeDtypeStruct((E, 128), jnp.float32),
            jax.ShapeDtypeStruct((NB, EB, 1), jnp.float32),
            jax.ShapeDtypeStruct((NB, EB, 1), jnp.int32),
            jax.ShapeDtypeStruct((G, 128), jnp.float32),
        ],
        scratch_shapes=[
            pltpu.VMEM((G, DH), jnp.float32),
            pltpu.VMEM((G, 1), jnp.float32),
        ],
        compiler_params=pltpu.CompilerParams(
            dimension_semantics=("arbitrary",)),
    )(yar, ybc, ea, seg16, wep, bep, w3, wsm, scal)


def _pass_body(bh_ref, segc_ref, st_ref, w3_ref, wsm_ref, scal_ref,
               bhout_ref, stout_ref, pooled_s):
    b = pl.program_id(0)

    @pl.when(b == 0)
    def _():
        pooled_s[...] = jnp.zeros_like(pooled_s)

    segc = segc_ref[0, :, :]
    oh = (segc == lax.broadcasted_iota(jnp.int32, (EB, G), 1)
          ).astype(jnp.float32)

    base = bh_ref[:, 0:64]
    h = bh_ref[:, 64:128]
    conf = st_ref[:, 0:1]
    lc = st_ref[:, 1:2]
    cur_p = jnp.tanh(base + jnp.dot(h, w3_ref[...],
                                    preferred_element_type=jnp.float32))
    coef_e = jnp.dot(oh, lc * conf, preferred_element_type=jnp.float32)
    hn = h + coef_e * cur_p
    bhout_ref[...] = jnp.concatenate([base, hn], axis=1)
    cur_n = jnp.tanh(base + jnp.dot(hn, w3_ref[...],
                                    preferred_element_type=jnp.float32))
    pooled_s[...] += lax.dot_general(oh, cur_n, _AtB,
                                     preferred_element_type=jnp.float32)

    @pl.when(b == NB - 1)
    def _():
        cnt = st_ref[:, 2:3]
        pooled_avg = pooled_s[...] / jnp.maximum(cnt, 1.0)
        wc = wsm_ref[:, 0:1]
        logits = jnp.dot(pooled_avg, wc, preferred_element_type=jnp.float32)
        conf_new = jax.nn.sigmoid(logits + scal_ref[0, 0])
        lc_new = lc * (1.0 - conf)
        cidx = lax.broadcasted_iota(jnp.int32, (G, 128), 1)
        stout_ref[...] = jnp.where(cidx == 0, conf_new,
                                   jnp.where(cidx == 1, lc_new,
                                             jnp.where(cidx == 2, cnt, 0.0)))


def _iter_pass(bh, segc, st, w3, wsm, scal):
    return pl.pallas_call(
        _pass_body,
        grid=(NB,),
        in_specs=[
            pl.BlockSpec((EB, 128), lambda b: (b, 0)),
            pl.BlockSpec((1, EB, 1), lambda b: (b, 0, 0)),
            pl.BlockSpec((G, 128), lambda b: (0, 0)),
            pl.BlockSpec((DH, DH), lambda b: (0, 0)),
            pl.BlockSpec((DH, 128), lambda b: (0, 0)),
            pl.BlockSpec(memory_space=pltpu.SMEM),
        ],
        out_specs=[
            pl.BlockSpec((EB, 128), lambda b: (b, 0)),
            pl.BlockSpec((G, 128), lambda b: (0, 0)),
        ],
        out_shape=[
            jax.ShapeDtypeStruct((E, 128), jnp.float32),
            jax.ShapeDtypeStruct((G, 128), jnp.float32),
        ],
        scratch_shapes=[
            pltpu.VMEM((G, DH), jnp.float32),
        ],
        compiler_params=pltpu.CompilerParams(
            dimension_semantics=("arbitrary",)),
    )(bh, segc, st, w3, wsm, scal)


def _readout_body(bh_ref, segc_ref, attc_ref, st_ref, w3_ref,
                  wsm_ref, scal_ref, out_ref, m_s, s_s, v_s):
    b = pl.program_id(0)

    @pl.when(b == 0)
    def _():
        m_s[...] = jnp.full_like(m_s, -1e30)
        s_s[...] = jnp.zeros_like(s_s)
        v_s[...] = jnp.zeros_like(v_s)

    segc = segc_ref[0, :, :]
    ohb = segc == lax.broadcasted_iota(jnp.int32, (EB, G), 1)
    oh = ohb.astype(jnp.float32)

    base = bh_ref[:, 0:64]
    h = bh_ref[:, 64:128]
    conf = st_ref[:, 0:1]
    lc = st_ref[:, 1:2]
    cur_p = jnp.tanh(base + jnp.dot(h, w3_ref[...],
                                    preferred_element_type=jnp.float32))
    coef_e = jnp.dot(oh, lc * conf, preferred_element_type=jnp.float32)
    hn = h + coef_e * cur_p

    wah = wsm_ref[:, 1:2]
    att = attc_ref[0, :, :] + jnp.dot(hn, wah,
                                      preferred_element_type=jnp.float32)
    masked = jnp.where(ohb, att, -1e30)
    m_blk = jnp.max(masked, axis=0, keepdims=True)
    m_new = jnp.maximum(m_s[...], m_blk)
    p = jnp.where(ohb, jnp.exp(att - m_new), 0.0)
    scale = jnp.exp(m_s[...] - m_new)
    s_s[...] = s_s[...] * scale + jnp.sum(p, axis=0, keepdims=True)
    v_s[...] = v_s[...] * scale + lax.dot_general(
        hn, p, _AtB, preferred_element_type=jnp.float32)
    m_s[...] = m_new

    @pl.when(b == NB - 1)
    def _():
        gf = v_s[...] / jnp.maximum(s_s[...], 1e-16)
        wh = wsm_ref[:, 2:3]
        out_ref[...] = lax.dot_general(
            wh, gf, _AtB, preferred_element_type=jnp.float32) + scal_ref[0, 1]


def _readout(bh, segc, attc, st, w3, wsm, scal):
    return pl.pallas_call(
        _readout_body,
        grid=(NB,),
        in_specs=[
            pl.BlockSpec((EB, 128), lambda b: (b, 0)),
            pl.BlockSpec((1, EB, 1), lambda b: (b, 0, 0)),
            pl.BlockSpec((1, EB, 1), lambda b: (b, 0, 0)),
            pl.BlockSpec((G, 128), lambda b: (0, 0)),
            pl.BlockSpec((DH, DH), lambda b: (0, 0)),
            pl.BlockSpec((DH, 128), lambda b: (0, 0)),
            pl.BlockSpec(memory_space=pltpu.SMEM),
        ],
        out_specs=pl.BlockSpec((1, G), lambda b: (0, 0)),
        out_shape=jax.ShapeDtypeStruct((1, G), jnp.float32),
        scratch_shapes=[
            pltpu.VMEM((1, G), jnp.float32),
            pltpu.VMEM((1, G), jnp.float32),
            pltpu.VMEM((DH, G), jnp.float32),
        ],
        compiler_params=pltpu.CompilerParams(
            dimension_semantics=("arbitrary",)),
    )(bh, segc, attc, st, w3, wsm, scal)


def kernel(x, edge_index, edge_attr, batch,
           W_emb, b_emb, W_body, b_body, W_conf, b_conf,
           W_att, b_att, W_head, b_head):
    rol = edge_index[0]
    col = edge_index[1]

    wa = jnp.zeros((DF, 80), jnp.float32)
    wa = wa.at[:, 0:64].set(W_body[0:DF]).at[:, 64].set(W_att[0:DF, 0])
    wb = jnp.zeros((DF, 80), jnp.float32)
    wb = wb.at[:, 0:64].set(W_body[DF:2 * DF]).at[:, 64].set(
        W_att[DF:2 * DF, 0])
    wab = jnp.concatenate([wa, wb], axis=1)

    wep = jnp.zeros((DE, 192), jnp.float32)
    wep = (wep.at[:, 0:64].set(W_body[2 * DF + DH:])
           .at[:, 64:128].set(W_emb)
           .at[:, 128].set(W_att[2 * DF + DH:, 0]))
    bep = jnp.zeros((1, 192), jnp.float32)
    bep = (bep.at[0, 0:64].set(b_body)
           .at[0, 64:128].set(b_emb)
           .at[0, 128].set(b_att[0]))

    w3 = W_body[2 * DF:2 * DF + DH]
    wsm = jnp.zeros((DH, 128), jnp.float32)
    wsm = (wsm.at[:, 0].set(W_conf[:, 0])
           .at[:, 1].set(W_att[2 * DF:2 * DF + DH, 0])
           .at[:, 2].set(W_head[:, 0]))
    scal = jnp.stack([b_conf[0], b_head[0]]).reshape(1, 2)

    batchc = batch.reshape(N // NBLK, NBLK, 1)
    ya, yb, node_counts, brep = _node_proj(x, batchc, wab)
    yar, ybc, seg16 = _edge_gather(ya, yb, brep, rol, col)
    bh, attc, segc, st = _edge_combine(yar, ybc, edge_attr, seg16,
                                       wep, bep, w3, wsm, scal)
    for _ in range(ITERS - 1):
        bh, st = _iter_pass(bh, segc, st, w3, wsm, scal)
    out_row = _readout(bh, segc, attc, st, w3, wsm, scal)
    return (out_row.reshape(G, 1), node_counts)


# speedup vs baseline: 1.2061x; 1.2061x over previous
import functools

import jax
import jax.numpy as jnp
from jax import lax
from jax.experimental import pallas as pl
from jax.experimental.pallas import tpu as pltpu
from jax.experimental.pallas import tpu_sc as plsc

N = 10000
E = 320000
DF = 128
DE = 16
DH = 64
G = 16
ITERS = 8

EB = 8000
EBC = 3200
NBC = E // EBC
NB = E // EB
NBLK = 1000
CH = 128
NCHUNK = E // CH
NW = 32

_AtB = (((0,), (0,)), ((), ()))


def _node_proj_body(x_ref, batchc_ref, wab_ref, ya_ref, yb_ref, cnt_ref,
                    brep_ref):
    b = pl.program_id(0)
    t = jnp.dot(x_ref[...], wab_ref[...], preferred_element_type=jnp.float32)
    ya_ref[...] = t[:, 0:80]
    yb_ref[...] = t[:, 80:160]
    bc = batchc_ref[0, :, :]
    brep_ref[...] = jnp.broadcast_to(bc, (NBLK, 16))
    oh = (bc == lax.broadcasted_iota(jnp.int32, (NBLK, G), 1)).astype(jnp.float32)
    ones = jnp.ones((NBLK, 1), dtype=jnp.float32)
    acc = lax.dot_general(oh, ones, _AtB, preferred_element_type=jnp.float32)

    @pl.when(b == 0)
    def _():
        cnt_ref[...] = jnp.zeros_like(cnt_ref)

    cnt_ref[...] += acc


def _node_proj(x, batchc, wab):
    nb = N // NBLK
    return pl.pallas_call(
        _node_proj_body,
        grid=(nb,),
        in_specs=[
            pl.BlockSpec((NBLK, DF), lambda b: (b, 0)),
            pl.BlockSpec((1, NBLK, 1), lambda b: (b, 0, 0)),
            pl.BlockSpec((DF, 160), lambda b: (0, 0)),
        ],
        out_specs=[
            pl.BlockSpec((NBLK, 80), lambda b: (b, 0)),
            pl.BlockSpec((NBLK, 80), lambda b: (b, 0)),
            pl.BlockSpec((G, 1), lambda b: (0, 0)),
            pl.BlockSpec((NBLK, 16), lambda b: (b, 0)),
        ],
        out_shape=[
            jax.ShapeDtypeStruct((N, 80), jnp.float32),
            jax.ShapeDtypeStruct((N, 80), jnp.float32),
            jax.ShapeDtypeStruct((G, 1), jnp.float32),
            jax.ShapeDtypeStruct((N, 16), jnp.int32),
        ],
        compiler_params=pltpu.CompilerParams(
            dimension_semantics=("arbitrary",)),
    )(x, batchc, wab)


def _gather_body(ya_hbm, yb_hbm, brep_hbm, rol_hbm, col_hbm,
                 yar_out, ybc_out, seg16_out,
                 idx_a, idx_b, rows_a, rows_b, rows_s, sem):
    cid = lax.axis_index("c")
    sid = lax.axis_index("s")
    wid = sid * 2 + cid
    nmine = (NCHUNK - wid + NW - 1) // NW

    def chunk(k, carry):
        c = wid + k * NW
        base = c * CH
        pltpu.sync_copy(rol_hbm.at[pl.ds(base, CH)], idx_a)
        pltpu.sync_copy(col_hbm.at[pl.ds(base, CH)], idx_b)
        cp_a = pltpu.async_copy(ya_hbm.at[idx_a], rows_a, sem)
        cp_b = pltpu.async_copy(yb_hbm.at[idx_b], rows_b, sem)
        cp_s = pltpu.async_copy(brep_hbm.at[idx_a], rows_s, sem)
        cp_a.wait()
        cp_b.wait()
        cp_s.wait()
        pltpu.sync_copy(rows_a, yar_out.at[pl.ds(base, CH)])
        pltpu.sync_copy(rows_b, ybc_out.at[pl.ds(base, CH)])
        pltpu.sync_copy(rows_s, seg16_out.at[pl.ds(base, CH)])
        return carry

    lax.fori_loop(0, nmine, chunk, 0)


def _edge_gather(ya, yb, brep, rol, col):
    mesh = plsc.VectorSubcoreMesh(core_axis_name="c", subcore_axis_name="s")
    fn = pl.kernel(
        _gather_body,
        out_type=[
            jax.ShapeDtypeStruct((E, 80), jnp.float32),
            jax.ShapeDtypeStruct((E, 80), jnp.float32),
            jax.ShapeDtypeStruct((E, 16), jnp.int32),
        ],
        mesh=mesh,
        scratch_types=[
            pltpu.VMEM((CH,), jnp.int32),
            pltpu.VMEM((CH,), jnp.int32),
            pltpu.VMEM((CH, 80), jnp.float32),
            pltpu.VMEM((CH, 80), jnp.float32),
            pltpu.VMEM((CH, 16), jnp.int32),
            pltpu.SemaphoreType.DMA,
        ],
        compiler_params=pltpu.CompilerParams(use_tc_tiling_on_sc=False),
    )
    return fn(ya, yb, brep, rol, col)


def _combine_body(yar_ref, ybc_ref, ea_ref, seg16_ref, wep_ref, bep_ref,
                  w3_ref, wsm_ref, scal_ref,
                  base_ref, hid_ref, attc_ref, segc_ref, st_ref,
                  pooled_s, cnt_s):
    b = pl.program_id(0)

    @pl.when(b == 0)
    def _():
        pooled_s[...] = jnp.zeros_like(pooled_s)
        cnt_s[...] = jnp.zeros_like(cnt_s)

    t = jnp.dot(ea_ref[...], wep_ref[...],
                preferred_element_type=jnp.float32) + bep_ref[...]
    base = yar_ref[:, 0:64] + ybc_ref[:, 0:64] + t[:, 0:64]
    hid0 = t[:, 64:128]
    base_ref[...] = base
    hid_ref[...] = hid0
    attc_ref[0, :, :] = (yar_ref[:, 64:65] + ybc_ref[:, 64:65]
                         + t[:, 128:129])
    segc = seg16_ref[:, 0:1]
    segc_ref[0, :, :] = segc

    oh = (segc == lax.broadcasted_iota(jnp.int32, (EBC, G), 1)
          ).astype(jnp.float32)
    cur0 = jnp.tanh(base + jnp.dot(hid0, w3_ref[...],
                                   preferred_element_type=jnp.float32))
    pooled_s[...] += lax.dot_general(oh, cur0, _AtB,
                                     preferred_element_type=jnp.float32)
    ones = jnp.ones((EBC, 1), dtype=jnp.float32)
    cnt_s[...] += lax.dot_general(oh, ones, _AtB,
                                  preferred_element_type=jnp.float32)

    @pl.when(b == NBC - 1)
    def _():
        cnt = cnt_s[...]
        pooled_avg = pooled_s[...] / jnp.maximum(cnt, 1.0)
        wc = wsm_ref[:, 0:1]
        logits = jnp.dot(pooled_avg, wc, preferred_element_type=jnp.float32)
        conf0 = jax.nn.sigmoid(logits + scal_ref[0, 0])
        cidx = lax.broadcasted_iota(jnp.int32, (G, 128), 1)
        st_ref[...] = jnp.where(cidx == 0, conf0,
                                jnp.where(cidx == 1, 1.0,
                                          jnp.where(cidx == 2, cnt, 0.0)))


def _edge_combine(yar, ybc, ea, seg16, wep, bep, w3, wsm, scal):
    return pl.pallas_call(
        _combine_body,
        grid=(NBC,),
        in_specs=[
            pl.BlockSpec((EBC, 80), lambda b: (b, 0)),
            pl.BlockSpec((EBC, 80), lambda b: (b, 0)),
            pl.BlockSpec((EBC, DE), lambda b: (b, 0)),
            pl.BlockSpec((EBC, 16), lambda b: (b, 0)),
            pl.BlockSpec((DE, 192), lambda b: (0, 0)),
            pl.BlockSpec((1, 192), lambda b: (0, 0)),
            pl.BlockSpec((DH, DH), lambda b: (0, 0)),
            pl.BlockSpec((DH, 128), lambda b: (0, 0)),
            pl.BlockSpec(memory_space=pltpu.SMEM),
        ],
        out_specs=[
            pl.BlockSpec((EBC, DH), lambda b: (b, 0)),
            pl.BlockSpec((EBC, DH), lambda b: (b, 0)),
            pl.BlockSpec((1, EBC, 1), lambda b: (b, 0, 0)),
            pl.BlockSpec((1, EBC, 1), lambda b: (b, 0, 0)),
            pl.BlockSpec((G, 128), lambda b: (0, 0)),
        ],
        out_shape=[
            jax.ShapeDtypeStruct((E, DH), jnp.float32),
            jax.ShapeDtypeStruct((E, DH), jnp.float32),
            jax.ShapeDtypeStruct((NBC, EBC, 1), jnp.float32),
            jax.ShapeDtypeStruct((NBC, EBC, 1), jnp.int32),
            jax.ShapeDtypeStruct((G, 128), jnp.float32),
        ],
        scratch_shapes=[
            pltpu.VMEM((G, DH), jnp.float32),
            pltpu.VMEM((G, 1), jnp.float32),
        ],
        compiler_params=pltpu.CompilerParams(
            dimension_semantics=("arbitrary",)),
    )(yar, ybc, ea, seg16, wep, bep, w3, wsm, scal)


def _pass_body(base_ref, hid_ref, segc_ref, st_ref, w3_ref, wsm_ref,
               scal_ref, hidout_ref, stout_ref, pooled_s):
    b = pl.program_id(0)

    @pl.when(b == 0)
    def _():
        pooled_s[...] = jnp.zeros_like(pooled_s)

    segc = segc_ref[0, :, :]
    oh = (segc == lax.broadcasted_iota(jnp.int32, (EB, G), 1)
          ).astype(jnp.float32)

    base = base_ref[...]
    h = hid_ref[...]
    conf = st_ref[:, 0:1]
    lc = st_ref[:, 1:2]
    cur_p = jnp.tanh(base + jnp.dot(h, w3_ref[...],
                                    preferred_element_type=jnp.float32))
    coef_e = jnp.dot(oh, lc * conf, preferred_element_type=jnp.float32)
    hn = h + coef_e * cur_p
    hidout_ref[...] = hn
    cur_n = jnp.tanh(base + jnp.dot(hn, w3_ref[...],
                                    preferred_element_type=jnp.float32))
    pooled_s[...] += lax.dot_general(oh, cur_n, _AtB,
                                     preferred_element_type=jnp.float32)

    @pl.when(b == NB - 1)
    def _():
        cnt = st_ref[:, 2:3]
        pooled_avg = pooled_s[...] / jnp.maximum(cnt, 1.0)
        wc = wsm_ref[:, 0:1]
        logits = jnp.dot(pooled_avg, wc, preferred_element_type=jnp.float32)
        conf_new = jax.nn.sigmoid(logits + scal_ref[0, 0])
        lc_new = lc * (1.0 - conf)
        cidx = lax.broadcasted_iota(jnp.int32, (G, 128), 1)
        stout_ref[...] = jnp.where(cidx == 0, conf_new,
                                   jnp.where(cidx == 1, lc_new,
                                             jnp.where(cidx == 2, cnt, 0.0)))


def _iter_pass(base, hid, segc, st, w3, wsm, scal):
    return pl.pallas_call(
        _pass_body,
        grid=(NB,),
        in_specs=[
            pl.BlockSpec((EB, DH), lambda b: (b, 0)),
            pl.BlockSpec((EB, DH), lambda b: (b, 0)),
            pl.BlockSpec((1, EB, 1), lambda b: (b, 0, 0)),
            pl.BlockSpec((G, 128), lambda b: (0, 0)),
            pl.BlockSpec((DH, DH), lambda b: (0, 0)),
            pl.BlockSpec((DH, 128), lambda b: (0, 0)),
            pl.BlockSpec(memory_space=pltpu.SMEM),
        ],
        out_specs=[
            pl.BlockSpec((EB, DH), lambda b: (b, 0)),
            pl.BlockSpec((G, 128), lambda b: (0, 0)),
        ],
        out_shape=[
            jax.ShapeDtypeStruct((E, DH), jnp.float32),
            jax.ShapeDtypeStruct((G, 128), jnp.float32),
        ],
        scratch_shapes=[
            pltpu.VMEM((G, DH), jnp.float32),
        ],
        compiler_params=pltpu.CompilerParams(
            dimension_semantics=("arbitrary",)),
    )(base, hid, segc, st, w3, wsm, scal)


def _readout_body(base_ref, hid_ref, segc_ref, attc_ref, st_ref, w3_ref,
                  wsm_ref, scal_ref, out_ref, m_s, s_s, v_s):
    b = pl.program_id(0)

    @pl.when(b == 0)
    def _():
        m_s[...] = jnp.full_like(m_s, -1e30)
        s_s[...] = jnp.zeros_like(s_s)
        v_s[...] = jnp.zeros_like(v_s)

    segc = segc_ref[0, :, :]
    ohb = segc == lax.broadcasted_iota(jnp.int32, (EB, G), 1)
    oh = ohb.astype(jnp.float32)

    base = base_ref[...]
    h = hid_ref[...]
    conf = st_ref[:, 0:1]
    lc = st_ref[:, 1:2]
    cur_p = jnp.tanh(base + jnp.dot(h, w3_ref[...],
                                    preferred_element_type=jnp.float32))
    coef_e = jnp.dot(oh, lc * conf, preferred_element_type=jnp.float32)
    hn = h + coef_e * cur_p

    wah = wsm_ref[:, 1:2]
    att = attc_ref[0, :, :] + jnp.dot(hn, wah,
                                      preferred_element_type=jnp.float32)
    masked = jnp.where(ohb, att, -1e30)
    m_blk = jnp.max(masked, axis=0, keepdims=True)
    m_new = jnp.maximum(m_s[...], m_blk)
    p = jnp.where(ohb, jnp.exp(att - m_new), 0.0)
    scale = jnp.exp(m_s[...] - m_new)
    s_s[...] = s_s[...] * scale + jnp.sum(p, axis=0, keepdims=True)
    v_s[...] = v_s[...] * scale + lax.dot_general(
        hn, p, _AtB, preferred_element_type=jnp.float32)
    m_s[...] = m_new

    @pl.when(b == NB - 1)
    def _():
        gf = v_s[...] / jnp.maximum(s_s[...], 1e-16)
        wh = wsm_ref[:, 2:3]
        out_ref[...] = lax.dot_general(
            wh, gf, _AtB, preferred_element_type=jnp.float32) + scal_ref[0, 1]


def _readout(base, hid, segc, attc, st, w3, wsm, scal):
    return pl.pallas_call(
        _readout_body,
        grid=(NB,),
        in_specs=[
            pl.BlockSpec((EB, DH), lambda b: (b, 0)),
            pl.BlockSpec((EB, DH), lambda b: (b, 0)),
            pl.BlockSpec((1, EB, 1), lambda b: (b, 0, 0)),
            pl.BlockSpec((1, EB, 1), lambda b: (b, 0, 0)),
            pl.BlockSpec((G, 128), lambda b: (0, 0)),
            pl.BlockSpec((DH, DH), lambda b: (0, 0)),
            pl.BlockSpec((DH, 128), lambda b: (0, 0)),
            pl.BlockSpec(memory_space=pltpu.SMEM),
        ],
        out_specs=pl.BlockSpec((1, G), lambda b: (0, 0)),
        out_shape=jax.ShapeDtypeStruct((1, G), jnp.float32),
        scratch_shapes=[
            pltpu.VMEM((1, G), jnp.float32),
            pltpu.VMEM((1, G), jnp.float32),
            pltpu.VMEM((DH, G), jnp.float32),
        ],
        compiler_params=pltpu.CompilerParams(
            dimension_semantics=("arbitrary",)),
    )(base, hid, segc, attc, st, w3, wsm, scal)


def kernel(x, edge_index, edge_attr, batch,
           W_emb, b_emb, W_body, b_body, W_conf, b_conf,
           W_att, b_att, W_head, b_head):
    rol = edge_index[0]
    col = edge_index[1]

    wa = jnp.zeros((DF, 80), jnp.float32)
    wa = wa.at[:, 0:64].set(W_body[0:DF]).at[:, 64].set(W_att[0:DF, 0])
    wb = jnp.zeros((DF, 80), jnp.float32)
    wb = wb.at[:, 0:64].set(W_body[DF:2 * DF]).at[:, 64].set(
        W_att[DF:2 * DF, 0])
    wab = jnp.concatenate([wa, wb], axis=1)

    wep = jnp.zeros((DE, 192), jnp.float32)
    wep = (wep.at[:, 0:64].set(W_body[2 * DF + DH:])
           .at[:, 64:128].set(W_emb)
           .at[:, 128].set(W_att[2 * DF + DH:, 0]))
    bep = jnp.zeros((1, 192), jnp.float32)
    bep = (bep.at[0, 0:64].set(b_body)
           .at[0, 64:128].set(b_emb)
           .at[0, 128].set(b_att[0]))

    w3 = W_body[2 * DF:2 * DF + DH]
    wsm = jnp.zeros((DH, 128), jnp.float32)
    wsm = (wsm.at[:, 0].set(W_conf[:, 0])
           .at[:, 1].set(W_att[2 * DF:2 * DF + DH, 0])
           .at[:, 2].set(W_head[:, 0]))
    scal = jnp.stack([b_conf[0], b_head[0]]).reshape(1, 2)

    batchc = batch.reshape(N // NBLK, NBLK, 1)
    ya, yb, node_counts, brep = _node_proj(x, batchc, wab)
    yar, ybc, seg16 = _edge_gather(ya, yb, brep, rol, col)
    base, hid, attc, segc, st = _edge_combine(yar, ybc, edge_attr, seg16,
                                              wep, bep, w3, wsm, scal)
    attc = attc.reshape(NB, EB, 1)
    segc = segc.reshape(NB, EB, 1)
    for _ in range(ITERS - 1):
        hid, st = _iter_pass(base, hid, segc, st, w3, wsm, scal)
    out_row = _readout(base, hid, segc, attc, st, w3, wsm, scal)
    return (out_row.reshape(G, 1), node_counts)


# speedup vs baseline: 1.3800x; 1.1442x over previous
import functools

import jax
import jax.numpy as jnp
from jax import lax
from jax.experimental import pallas as pl
from jax.experimental.pallas import tpu as pltpu
from jax.experimental.pallas import tpu_sc as plsc

N = 10000
E = 320000
DF = 128
DE = 16
DH = 64
G = 16
ITERS = 8

EB = 8000
EBC = 3200
NBC = E // EBC
NB = E // EB
NBLK = 1000
CH = 128
NCHUNK = E // CH
NW = 32

_AtB = (((0,), (0,)), ((), ()))


def _node_proj_body(x_ref, batchc_ref, wab_ref, ya_ref, yb_ref, cnt_ref,
                    brep_ref):
    b = pl.program_id(0)
    t = jnp.dot(x_ref[...], wab_ref[...], preferred_element_type=jnp.float32)
    ya_ref[...] = t[:, 0:80]
    yb_ref[...] = t[:, 80:160]
    bc = batchc_ref[0, :, :]
    brep_ref[...] = jnp.broadcast_to(bc, (NBLK, 16))
    oh = (bc == lax.broadcasted_iota(jnp.int32, (NBLK, G), 1)).astype(jnp.float32)
    ones = jnp.ones((NBLK, 1), dtype=jnp.float32)
    acc = lax.dot_general(oh, ones, _AtB, preferred_element_type=jnp.float32)

    @pl.when(b == 0)
    def _():
        cnt_ref[...] = jnp.zeros_like(cnt_ref)

    cnt_ref[...] += acc


def _node_proj(x, batchc, wab):
    nb = N // NBLK
    return pl.pallas_call(
        _node_proj_body,
        grid=(nb,),
        in_specs=[
            pl.BlockSpec((NBLK, DF), lambda b: (b, 0)),
            pl.BlockSpec((1, NBLK, 1), lambda b: (b, 0, 0)),
            pl.BlockSpec((DF, 160), lambda b: (0, 0)),
        ],
        out_specs=[
            pl.BlockSpec((NBLK, 80), lambda b: (b, 0)),
            pl.BlockSpec((NBLK, 80), lambda b: (b, 0)),
            pl.BlockSpec((G, 1), lambda b: (0, 0)),
            pl.BlockSpec((NBLK, 16), lambda b: (b, 0)),
        ],
        out_shape=[
            jax.ShapeDtypeStruct((N, 80), jnp.float32),
            jax.ShapeDtypeStruct((N, 80), jnp.float32),
            jax.ShapeDtypeStruct((G, 1), jnp.float32),
            jax.ShapeDtypeStruct((N, 16), jnp.int32),
        ],
        compiler_params=pltpu.CompilerParams(
            dimension_semantics=("arbitrary",)),
    )(x, batchc, wab)


def _gather_body(ya_hbm, yb_hbm, brep_hbm, rol_hbm, col_hbm,
                 yar_out, ybc_out, seg16_out,
                 idx_a, idx_b, rows_a, rows_b, rows_s, sem):
    cid = lax.axis_index("c")
    sid = lax.axis_index("s")
    wid = sid * 2 + cid
    nmine = (NCHUNK - wid + NW - 1) // NW

    def chunk(k, carry):
        c = wid + k * NW
        base = c * CH
        pltpu.sync_copy(rol_hbm.at[pl.ds(base, CH)], idx_a)
        pltpu.sync_copy(col_hbm.at[pl.ds(base, CH)], idx_b)
        cp_a = pltpu.async_copy(ya_hbm.at[idx_a], rows_a, sem)
        cp_b = pltpu.async_copy(yb_hbm.at[idx_b], rows_b, sem)
        cp_s = pltpu.async_copy(brep_hbm.at[idx_a], rows_s, sem)
        cp_a.wait()
        cp_b.wait()
        cp_s.wait()
        pltpu.sync_copy(rows_a, yar_out.at[pl.ds(base, CH)])
        pltpu.sync_copy(rows_b, ybc_out.at[pl.ds(base, CH)])
        pltpu.sync_copy(rows_s, seg16_out.at[pl.ds(base, CH)])
        return carry

    lax.fori_loop(0, nmine, chunk, 0)


def _edge_gather(ya, yb, brep, rol, col):
    mesh = plsc.VectorSubcoreMesh(core_axis_name="c", subcore_axis_name="s")
    fn = pl.kernel(
        _gather_body,
        out_type=[
            jax.ShapeDtypeStruct((E, 80), jnp.float32),
            jax.ShapeDtypeStruct((E, 80), jnp.float32),
            jax.ShapeDtypeStruct((E, 16), jnp.int32),
        ],
        mesh=mesh,
        scratch_types=[
            pltpu.VMEM((CH,), jnp.int32),
            pltpu.VMEM((CH,), jnp.int32),
            pltpu.VMEM((CH, 80), jnp.float32),
            pltpu.VMEM((CH, 80), jnp.float32),
            pltpu.VMEM((CH, 16), jnp.int32),
            pltpu.SemaphoreType.DMA,
        ],
        compiler_params=pltpu.CompilerParams(use_tc_tiling_on_sc=False),
    )
    return fn(ya, yb, brep, rol, col)


def _combine_body(yar_ref, ybc_ref, ea_ref, seg16_ref, wep_ref, bep_ref,
                  w3_ref, wsm_ref, scal_ref,
                  base_ref, hid_ref, attc_ref, segc_ref, st_ref,
                  pooled_s, cnt_s):
    b = pl.program_id(0)

    @pl.when(b == 0)
    def _():
        pooled_s[...] = jnp.zeros_like(pooled_s)
        cnt_s[...] = jnp.zeros_like(cnt_s)

    t = jnp.dot(ea_ref[...], wep_ref[...],
                preferred_element_type=jnp.float32) + bep_ref[...]
    base = yar_ref[:, 0:64] + ybc_ref[:, 0:64] + t[:, 0:64]
    hid0 = t[:, 64:128]
    base_ref[...] = base.astype(jnp.bfloat16)
    hid_ref[...] = hid0.astype(jnp.bfloat16)
    attc_ref[0, :, :] = (yar_ref[:, 64:65] + ybc_ref[:, 64:65]
                         + t[:, 128:129])
    segc = seg16_ref[:, 0:1]
    segc_ref[0, :, :] = segc

    oh = (segc == lax.broadcasted_iota(jnp.int32, (EBC, G), 1)
          ).astype(jnp.float32)
    cur0 = jnp.tanh(base + jnp.dot(hid0, w3_ref[...],
                                   preferred_element_type=jnp.float32))
    pooled_s[...] += lax.dot_general(oh, cur0, _AtB,
                                     preferred_element_type=jnp.float32)
    ones = jnp.ones((EBC, 1), dtype=jnp.float32)
    cnt_s[...] += lax.dot_general(oh, ones, _AtB,
                                  preferred_element_type=jnp.float32)

    @pl.when(b == NBC - 1)
    def _():
        cnt = cnt_s[...]
        pooled_avg = pooled_s[...] / jnp.maximum(cnt, 1.0)
        wc = wsm_ref[:, 0:1]
        logits = jnp.dot(pooled_avg, wc, preferred_element_type=jnp.float32)
        conf0 = jax.nn.sigmoid(logits + scal_ref[0, 0])
        cidx = lax.broadcasted_iota(jnp.int32, (G, 128), 1)
        st_ref[...] = jnp.where(cidx == 0, conf0,
                                jnp.where(cidx == 1, 1.0,
                                          jnp.where(cidx == 2, cnt, 0.0)))


def _edge_combine(yar, ybc, ea, seg16, wep, bep, w3, wsm, scal):
    return pl.pallas_call(
        _combine_body,
        grid=(NBC,),
        in_specs=[
            pl.BlockSpec((EBC, 80), lambda b: (b, 0)),
            pl.BlockSpec((EBC, 80), lambda b: (b, 0)),
            pl.BlockSpec((EBC, DE), lambda b: (b, 0)),
            pl.BlockSpec((EBC, 16), lambda b: (b, 0)),
            pl.BlockSpec((DE, 192), lambda b: (0, 0)),
            pl.BlockSpec((1, 192), lambda b: (0, 0)),
            pl.BlockSpec((DH, DH), lambda b: (0, 0)),
            pl.BlockSpec((DH, 128), lambda b: (0, 0)),
            pl.BlockSpec(memory_space=pltpu.SMEM),
        ],
        out_specs=[
            pl.BlockSpec((EBC, DH), lambda b: (b, 0)),
            pl.BlockSpec((EBC, DH), lambda b: (b, 0)),
            pl.BlockSpec((1, EBC, 1), lambda b: (b, 0, 0)),
            pl.BlockSpec((1, EBC, 1), lambda b: (b, 0, 0)),
            pl.BlockSpec((G, 128), lambda b: (0, 0)),
        ],
        out_shape=[
            jax.ShapeDtypeStruct((E, DH), jnp.bfloat16),
            jax.ShapeDtypeStruct((E, DH), jnp.bfloat16),
            jax.ShapeDtypeStruct((NBC, EBC, 1), jnp.float32),
            jax.ShapeDtypeStruct((NBC, EBC, 1), jnp.int32),
            jax.ShapeDtypeStruct((G, 128), jnp.float32),
        ],
        scratch_shapes=[
            pltpu.VMEM((G, DH), jnp.float32),
            pltpu.VMEM((G, 1), jnp.float32),
        ],
        compiler_params=pltpu.CompilerParams(
            dimension_semantics=("arbitrary",)),
    )(yar, ybc, ea, seg16, wep, bep, w3, wsm, scal)


def _pass_body(base_ref, hid_ref, segc_ref, st_ref, w3_ref, wsm_ref,
               scal_ref, hidout_ref, stout_ref, pooled_s):
    b = pl.program_id(0)

    @pl.when(b == 0)
    def _():
        pooled_s[...] = jnp.zeros_like(pooled_s)

    segc = segc_ref[0, :, :]
    oh = (segc == lax.broadcasted_iota(jnp.int32, (EB, G), 1)
          ).astype(jnp.float32)

    base = base_ref[...].astype(jnp.float32)
    h = hid_ref[...].astype(jnp.float32)
    conf = st_ref[:, 0:1]
    lc = st_ref[:, 1:2]
    cur_p = jnp.tanh(base + jnp.dot(h, w3_ref[...],
                                    preferred_element_type=jnp.float32))
    coef_e = jnp.dot(oh, lc * conf, preferred_element_type=jnp.float32)
    hn = h + coef_e * cur_p
    hidout_ref[...] = hn.astype(jnp.bfloat16)
    cur_n = jnp.tanh(base + jnp.dot(hn, w3_ref[...],
                                    preferred_element_type=jnp.float32))
    pooled_s[...] += lax.dot_general(oh, cur_n, _AtB,
                                     preferred_element_type=jnp.float32)

    @pl.when(b == NB - 1)
    def _():
        cnt = st_ref[:, 2:3]
        pooled_avg = pooled_s[...] / jnp.maximum(cnt, 1.0)
        wc = wsm_ref[:, 0:1]
        logits = jnp.dot(pooled_avg, wc, preferred_element_type=jnp.float32)
        conf_new = jax.nn.sigmoid(logits + scal_ref[0, 0])
        lc_new = lc * (1.0 - conf)
        cidx = lax.broadcasted_iota(jnp.int32, (G, 128), 1)
        stout_ref[...] = jnp.where(cidx == 0, conf_new,
                                   jnp.where(cidx == 1, lc_new,
                                             jnp.where(cidx == 2, cnt, 0.0)))


def _iter_pass(base, hid, segc, st, w3, wsm, scal):
    return pl.pallas_call(
        _pass_body,
        grid=(NB,),
        in_specs=[
            pl.BlockSpec((EB, DH), lambda b: (b, 0)),
            pl.BlockSpec((EB, DH), lambda b: (b, 0)),
            pl.BlockSpec((1, EB, 1), lambda b: (b, 0, 0)),
            pl.BlockSpec((G, 128), lambda b: (0, 0)),
            pl.BlockSpec((DH, DH), lambda b: (0, 0)),
            pl.BlockSpec((DH, 128), lambda b: (0, 0)),
            pl.BlockSpec(memory_space=pltpu.SMEM),
        ],
        out_specs=[
            pl.BlockSpec((EB, DH), lambda b: (b, 0)),
            pl.BlockSpec((G, 128), lambda b: (0, 0)),
        ],
        out_shape=[
            jax.ShapeDtypeStruct((E, DH), jnp.bfloat16),
            jax.ShapeDtypeStruct((G, 128), jnp.float32),
        ],
        scratch_shapes=[
            pltpu.VMEM((G, DH), jnp.float32),
        ],
        compiler_params=pltpu.CompilerParams(
            dimension_semantics=("arbitrary",)),
    )(base, hid, segc, st, w3, wsm, scal)


def _readout_body(base_ref, hid_ref, segc_ref, attc_ref, st_ref, w3_ref,
                  wsm_ref, scal_ref, out_ref, m_s, s_s, v_s):
    b = pl.program_id(0)

    @pl.when(b == 0)
    def _():
        m_s[...] = jnp.full_like(m_s, -1e30)
        s_s[...] = jnp.zeros_like(s_s)
        v_s[...] = jnp.zeros_like(v_s)

    segc = segc_ref[0, :, :]
    ohb = segc == lax.broadcasted_iota(jnp.int32, (EB, G), 1)
    oh = ohb.astype(jnp.float32)

    base = base_ref[...].astype(jnp.float32)
    h = hid_ref[...].astype(jnp.float32)
    conf = st_ref[:, 0:1]
    lc = st_ref[:, 1:2]
    cur_p = jnp.tanh(base + jnp.dot(h, w3_ref[...],
                                    preferred_element_type=jnp.float32))
    coef_e = jnp.dot(oh, lc * conf, preferred_element_type=jnp.float32)
    hn = h + coef_e * cur_p

    wah = wsm_ref[:, 1:2]
    att = attc_ref[0, :, :] + jnp.dot(hn, wah,
                                      preferred_element_type=jnp.float32)
    masked = jnp.where(ohb, att, -1e30)
    m_blk = jnp.max(masked, axis=0, keepdims=True)
    m_new = jnp.maximum(m_s[...], m_blk)
    p = jnp.where(ohb, jnp.exp(att - m_new), 0.0)
    scale = jnp.exp(m_s[...] - m_new)
    s_s[...] = s_s[...] * scale + jnp.sum(p, axis=0, keepdims=True)
    v_s[...] = v_s[...] * scale + lax.dot_general(
        hn, p, _AtB, preferred_element_type=jnp.float32)
    m_s[...] = m_new

    @pl.when(b == NB - 1)
    def _():
        gf = v_s[...] / jnp.maximum(s_s[...], 1e-16)
        wh = wsm_ref[:, 2:3]
        out_ref[...] = lax.dot_general(
            wh, gf, _AtB, preferred_element_type=jnp.float32) + scal_ref[0, 1]


def _readout(base, hid, segc, attc, st, w3, wsm, scal):
    return pl.pallas_call(
        _readout_body,
        grid=(NB,),
        in_specs=[
            pl.BlockSpec((EB, DH), lambda b: (b, 0)),
            pl.BlockSpec((EB, DH), lambda b: (b, 0)),
            pl.BlockSpec((1, EB, 1), lambda b: (b, 0, 0)),
            pl.BlockSpec((1, EB, 1), lambda b: (b, 0, 0)),
            pl.BlockSpec((G, 128), lambda b: (0, 0)),
            pl.BlockSpec((DH, DH), lambda b: (0, 0)),
            pl.BlockSpec((DH, 128), lambda b: (0, 0)),
            pl.BlockSpec(memory_space=pltpu.SMEM),
        ],
        out_specs=pl.BlockSpec((1, G), lambda b: (0, 0)),
        out_shape=jax.ShapeDtypeStruct((1, G), jnp.float32),
        scratch_shapes=[
            pltpu.VMEM((1, G), jnp.float32),
            pltpu.VMEM((1, G), jnp.float32),
            pltpu.VMEM((DH, G), jnp.float32),
        ],
        compiler_params=pltpu.CompilerParams(
            dimension_semantics=("arbitrary",)),
    )(base, hid, segc, attc, st, w3, wsm, scal)


def kernel(x, edge_index, edge_attr, batch,
           W_emb, b_emb, W_body, b_body, W_conf, b_conf,
           W_att, b_att, W_head, b_head):
    rol = edge_index[0]
    col = edge_index[1]

    wa = jnp.zeros((DF, 80), jnp.float32)
    wa = wa.at[:, 0:64].set(W_body[0:DF]).at[:, 64].set(W_att[0:DF, 0])
    wb = jnp.zeros((DF, 80), jnp.float32)
    wb = wb.at[:, 0:64].set(W_body[DF:2 * DF]).at[:, 64].set(
        W_att[DF:2 * DF, 0])
    wab = jnp.concatenate([wa, wb], axis=1)

    wep = jnp.zeros((DE, 192), jnp.float32)
    wep = (wep.at[:, 0:64].set(W_body[2 * DF + DH:])
           .at[:, 64:128].set(W_emb)
           .at[:, 128].set(W_att[2 * DF + DH:, 0]))
    bep = jnp.zeros((1, 192), jnp.float32)
    bep = (bep.at[0, 0:64].set(b_body)
           .at[0, 64:128].set(b_emb)
           .at[0, 128].set(b_att[0]))

    w3 = W_body[2 * DF:2 * DF + DH]
    wsm = jnp.zeros((DH, 128), jnp.float32)
    wsm = (wsm.at[:, 0].set(W_conf[:, 0])
           .at[:, 1].set(W_att[2 * DF:2 * DF + DH, 0])
           .at[:, 2].set(W_head[:, 0]))
    scal = jnp.stack([b_conf[0], b_head[0]]).reshape(1, 2)

    batchc = batch.reshape(N // NBLK, NBLK, 1)
    ya, yb, node_counts, brep = _node_proj(x, batchc, wab)
    yar, ybc, seg16 = _edge_gather(ya, yb, brep, rol, col)
    base, hid, attc, segc, st = _edge_combine(yar, ybc, edge_attr, seg16,
                                              wep, bep, w3, wsm, scal)
    attc = attc.reshape(NB, EB, 1)
    segc = segc.reshape(NB, EB, 1)
    for _ in range(ITERS - 1):
        hid, st = _iter_pass(base, hid, segc, st, w3, wsm, scal)
    out_row = _readout(base, hid, segc, attc, st, w3, wsm, scal)
    return (out_row.reshape(G, 1), node_counts)


# speedup vs baseline: 1.4836x; 1.0751x over previous
import functools

import jax
import jax.numpy as jnp
from jax import lax
from jax.experimental import pallas as pl
from jax.experimental.pallas import tpu as pltpu
from jax.experimental.pallas import tpu_sc as plsc

N = 10000
E = 320000
DF = 128
DE = 16
DH = 64
G = 16
ITERS = 8

EB = 8000
EBC = 3200
NBC = E // EBC
NB = E // EB
NBLK = 1000
CH = 128
NCHUNK = E // CH
NW = 32

_AtB = (((0,), (0,)), ((), ()))


def _node_proj_body(x_ref, batchc_ref, wab_ref, ya_ref, yb_ref, cnt_ref,
                    brep_ref):
    b = pl.program_id(0)
    t = jnp.dot(x_ref[...], wab_ref[...], preferred_element_type=jnp.float32)
    ya_ref[...] = t[:, 0:80]
    yb_ref[...] = t[:, 80:160]
    bc = batchc_ref[0, :, :]
    brep_ref[...] = jnp.broadcast_to(bc, (NBLK, 16))
    oh = (bc == lax.broadcasted_iota(jnp.int32, (NBLK, G), 1)).astype(jnp.float32)
    ones = jnp.ones((NBLK, 1), dtype=jnp.float32)
    acc = lax.dot_general(oh, ones, _AtB, preferred_element_type=jnp.float32)

    @pl.when(b == 0)
    def _():
        cnt_ref[...] = jnp.zeros_like(cnt_ref)

    cnt_ref[...] += acc


def _node_proj(x, batchc, wab):
    nb = N // NBLK
    return pl.pallas_call(
        _node_proj_body,
        grid=(nb,),
        in_specs=[
            pl.BlockSpec((NBLK, DF), lambda b: (b, 0)),
            pl.BlockSpec((1, NBLK, 1), lambda b: (b, 0, 0)),
            pl.BlockSpec((DF, 160), lambda b: (0, 0)),
        ],
        out_specs=[
            pl.BlockSpec((NBLK, 80), lambda b: (b, 0)),
            pl.BlockSpec((NBLK, 80), lambda b: (b, 0)),
            pl.BlockSpec((G, 1), lambda b: (0, 0)),
            pl.BlockSpec((NBLK, 16), lambda b: (b, 0)),
        ],
        out_shape=[
            jax.ShapeDtypeStruct((N, 80), jnp.float32),
            jax.ShapeDtypeStruct((N, 80), jnp.float32),
            jax.ShapeDtypeStruct((G, 1), jnp.float32),
            jax.ShapeDtypeStruct((N, 16), jnp.int32),
        ],
        compiler_params=pltpu.CompilerParams(
            dimension_semantics=("arbitrary",)),
    )(x, batchc, wab)


def _gather_body(ya_hbm, yb_hbm, brep_hbm, rol_hbm, col_hbm,
                 yar_out, ybc_out, seg16_out,
                 idx_a, idx_b, rows_a, rows_b, rows_s, sem):
    cid = lax.axis_index("c")
    sid = lax.axis_index("s")
    wid = sid * 2 + cid
    nmine = (NCHUNK - wid + NW - 1) // NW

    def chunk(k, carry):
        c = wid + k * NW
        base = c * CH
        pltpu.sync_copy(rol_hbm.at[pl.ds(base, CH)], idx_a)
        pltpu.sync_copy(col_hbm.at[pl.ds(base, CH)], idx_b)
        cp_a = pltpu.async_copy(ya_hbm.at[idx_a], rows_a, sem)
        cp_b = pltpu.async_copy(yb_hbm.at[idx_b], rows_b, sem)
        cp_s = pltpu.async_copy(brep_hbm.at[idx_a], rows_s, sem)
        cp_a.wait()
        cp_b.wait()
        cp_s.wait()
        pltpu.sync_copy(rows_a, yar_out.at[pl.ds(base, CH)])
        pltpu.sync_copy(rows_b, ybc_out.at[pl.ds(base, CH)])
        pltpu.sync_copy(rows_s, seg16_out.at[pl.ds(base, CH)])
        return carry

    lax.fori_loop(0, nmine, chunk, 0)


def _edge_gather(ya, yb, brep, rol, col):
    mesh = plsc.VectorSubcoreMesh(core_axis_name="c", subcore_axis_name="s")
    fn = pl.kernel(
        _gather_body,
        out_type=[
            jax.ShapeDtypeStruct((E, 80), jnp.float32),
            jax.ShapeDtypeStruct((E, 80), jnp.float32),
            jax.ShapeDtypeStruct((E, 16), jnp.int32),
        ],
        mesh=mesh,
        scratch_types=[
            pltpu.VMEM((CH,), jnp.int32),
            pltpu.VMEM((CH,), jnp.int32),
            pltpu.VMEM((CH, 80), jnp.float32),
            pltpu.VMEM((CH, 80), jnp.float32),
            pltpu.VMEM((CH, 16), jnp.int32),
            pltpu.SemaphoreType.DMA,
        ],
        compiler_params=pltpu.CompilerParams(use_tc_tiling_on_sc=False),
    )
    return fn(ya, yb, brep, rol, col)


def _combine_body(yar_ref, ybc_ref, ea_ref, seg16_ref, wep_ref, bep_ref,
                  w3_ref, wsm_ref, scal_ref,
                  base_ref, hid_ref, attc_ref, segc_ref, st_ref,
                  pooled_s, cnt_s):
    b = pl.program_id(0)

    @pl.when(b == 0)
    def _():
        pooled_s[...] = jnp.zeros_like(pooled_s)
        cnt_s[...] = jnp.zeros_like(cnt_s)

    t = jnp.dot(ea_ref[...], wep_ref[...],
                preferred_element_type=jnp.float32) + bep_ref[...]
    base = yar_ref[:, 0:64] + ybc_ref[:, 0:64] + t[:, 0:64]
    hid0 = t[:, 64:128]
    base_ref[...] = base.astype(jnp.bfloat16)
    hid_ref[...] = hid0.astype(jnp.bfloat16)
    attc_ref[0, :, :] = (yar_ref[:, 64:65] + ybc_ref[:, 64:65]
                         + t[:, 128:129])
    segc = seg16_ref[:, 0:1]
    segc_ref[0, :, :] = segc.astype(jnp.float32).astype(jnp.bfloat16)

    oh = (segc == lax.broadcasted_iota(jnp.int32, (EBC, G), 1)
          ).astype(jnp.float32)
    cur0 = jnp.tanh(base + jnp.dot(hid0, w3_ref[...],
                                   preferred_element_type=jnp.float32))
    pooled_s[...] += lax.dot_general(oh, cur0, _AtB,
                                     preferred_element_type=jnp.float32)
    ones = jnp.ones((EBC, 1), dtype=jnp.float32)
    cnt_s[...] += lax.dot_general(oh, ones, _AtB,
                                  preferred_element_type=jnp.float32)

    @pl.when(b == NBC - 1)
    def _():
        cnt = cnt_s[...]
        pooled_avg = pooled_s[...] / jnp.maximum(cnt, 1.0)
        wc = wsm_ref[:, 0:1]
        logits = jnp.dot(pooled_avg, wc, preferred_element_type=jnp.float32)
        conf0 = jax.nn.sigmoid(logits + scal_ref[0, 0])
        cidx = lax.broadcasted_iota(jnp.int32, (G, 128), 1)
        st_ref[...] = jnp.where(cidx == 0, conf0,
                                jnp.where(cidx == 1, 1.0,
                                          jnp.where(cidx == 2, cnt, 0.0)))


def _edge_combine(yar, ybc, ea, seg16, wep, bep, w3, wsm, scal):
    return pl.pallas_call(
        _combine_body,
        grid=(NBC,),
        in_specs=[
            pl.BlockSpec((EBC, 80), lambda b: (b, 0)),
            pl.BlockSpec((EBC, 80), lambda b: (b, 0)),
            pl.BlockSpec((EBC, DE), lambda b: (b, 0)),
            pl.BlockSpec((EBC, 16), lambda b: (b, 0)),
            pl.BlockSpec((DE, 192), lambda b: (0, 0)),
            pl.BlockSpec((1, 192), lambda b: (0, 0)),
            pl.BlockSpec((DH, DH), lambda b: (0, 0)),
            pl.BlockSpec((DH, 128), lambda b: (0, 0)),
            pl.BlockSpec(memory_space=pltpu.SMEM),
        ],
        out_specs=[
            pl.BlockSpec((EBC, DH), lambda b: (b, 0)),
            pl.BlockSpec((EBC, DH), lambda b: (b, 0)),
            pl.BlockSpec((1, EBC, 1), lambda b: (b, 0, 0)),
            pl.BlockSpec((1, EBC, 1), lambda b: (b, 0, 0)),
            pl.BlockSpec((G, 128), lambda b: (0, 0)),
        ],
        out_shape=[
            jax.ShapeDtypeStruct((E, DH), jnp.bfloat16),
            jax.ShapeDtypeStruct((E, DH), jnp.bfloat16),
            jax.ShapeDtypeStruct((NBC, EBC, 1), jnp.float32),
            jax.ShapeDtypeStruct((NBC, EBC, 1), jnp.bfloat16),
            jax.ShapeDtypeStruct((G, 128), jnp.float32),
        ],
        scratch_shapes=[
            pltpu.VMEM((G, DH), jnp.float32),
            pltpu.VMEM((G, 1), jnp.float32),
        ],
        compiler_params=pltpu.CompilerParams(
            dimension_semantics=("arbitrary",)),
    )(yar, ybc, ea, seg16, wep, bep, w3, wsm, scal)


def _pass_body(base_ref, hid_ref, segc_ref, st_ref, w3_ref, wsm_ref,
               scal_ref, hidout_ref, stout_ref, pooled_s):
    b = pl.program_id(0)

    @pl.when(b == 0)
    def _():
        pooled_s[...] = jnp.zeros_like(pooled_s)

    segc = segc_ref[0, :, :].astype(jnp.float32)
    oh = (segc == lax.broadcasted_iota(jnp.int32, (EB, G), 1)
          .astype(jnp.float32)).astype(jnp.float32)

    base = base_ref[...].astype(jnp.float32)
    h = hid_ref[...].astype(jnp.float32)
    conf = st_ref[:, 0:1]
    lc = st_ref[:, 1:2]
    cur_p = jnp.tanh(base + jnp.dot(h, w3_ref[...],
                                    preferred_element_type=jnp.float32))
    coef_e = jnp.dot(oh, lc * conf, preferred_element_type=jnp.float32)
    hn = h + coef_e * cur_p
    hidout_ref[...] = hn.astype(jnp.bfloat16)
    cur_n = jnp.tanh(base + jnp.dot(hn, w3_ref[...],
                                    preferred_element_type=jnp.float32))
    pooled_s[...] += lax.dot_general(oh, cur_n, _AtB,
                                     preferred_element_type=jnp.float32)

    @pl.when(b == NB - 1)
    def _():
        cnt = st_ref[:, 2:3]
        pooled_avg = pooled_s[...] / jnp.maximum(cnt, 1.0)
        wc = wsm_ref[:, 0:1]
        logits = jnp.dot(pooled_avg, wc, preferred_element_type=jnp.float32)
        conf_new = jax.nn.sigmoid(logits + scal_ref[0, 0])
        lc_new = lc * (1.0 - conf)
        cidx = lax.broadcasted_iota(jnp.int32, (G, 128), 1)
        stout_ref[...] = jnp.where(cidx == 0, conf_new,
                                   jnp.where(cidx == 1, lc_new,
                                             jnp.where(cidx == 2, cnt, 0.0)))


def _iter_pass(base, hid, segc, st, w3, wsm, scal):
    return pl.pallas_call(
        _pass_body,
        grid=(NB,),
        in_specs=[
            pl.BlockSpec((EB, DH), lambda b: (b, 0)),
            pl.BlockSpec((EB, DH), lambda b: (b, 0)),
            pl.BlockSpec((1, EB, 1), lambda b: (b, 0, 0)),
            pl.BlockSpec((G, 128), lambda b: (0, 0)),
            pl.BlockSpec((DH, DH), lambda b: (0, 0)),
            pl.BlockSpec((DH, 128), lambda b: (0, 0)),
            pl.BlockSpec(memory_space=pltpu.SMEM),
        ],
        out_specs=[
            pl.BlockSpec((EB, DH), lambda b: (b, 0)),
            pl.BlockSpec((G, 128), lambda b: (0, 0)),
        ],
        out_shape=[
            jax.ShapeDtypeStruct((E, DH), jnp.bfloat16),
            jax.ShapeDtypeStruct((G, 128), jnp.float32),
        ],
        scratch_shapes=[
            pltpu.VMEM((G, DH), jnp.float32),
        ],
        compiler_params=pltpu.CompilerParams(
            dimension_semantics=("arbitrary",)),
    )(base, hid, segc, st, w3, wsm, scal)


def _readout_body(base_ref, hid_ref, segc_ref, attc_ref, st_ref, w3_ref,
                  wsm_ref, scal_ref, out_ref, m_s, s_s, v_s):
    b = pl.program_id(0)

    @pl.when(b == 0)
    def _():
        m_s[...] = jnp.full_like(m_s, -1e30)
        s_s[...] = jnp.zeros_like(s_s)
        v_s[...] = jnp.zeros_like(v_s)

    segc = segc_ref[0, :, :].astype(jnp.float32)
    ohb = segc == lax.broadcasted_iota(jnp.int32, (EB, G), 1).astype(
        jnp.float32)
    oh = ohb.astype(jnp.float32)

    base = base_ref[...].astype(jnp.float32)
    h = hid_ref[...].astype(jnp.float32)
    conf = st_ref[:, 0:1]
    lc = st_ref[:, 1:2]
    cur_p = jnp.tanh(base + jnp.dot(h, w3_ref[...],
                                    preferred_element_type=jnp.float32))
    coef_e = jnp.dot(oh, lc * conf, preferred_element_type=jnp.float32)
    hn = h + coef_e * cur_p

    wah = wsm_ref[:, 1:2]
    att = attc_ref[0, :, :] + jnp.dot(hn, wah,
                                      preferred_element_type=jnp.float32)
    masked = jnp.where(ohb, att, -1e30)
    m_blk = jnp.max(masked, axis=0, keepdims=True)
    m_new = jnp.maximum(m_s[...], m_blk)
    p = jnp.where(ohb, jnp.exp(att - m_new), 0.0)
    scale = jnp.exp(m_s[...] - m_new)
    s_s[...] = s_s[...] * scale + jnp.sum(p, axis=0, keepdims=True)
    v_s[...] = v_s[...] * scale + lax.dot_general(
        hn, p, _AtB, preferred_element_type=jnp.float32)
    m_s[...] = m_new

    @pl.when(b == NB - 1)
    def _():
        gf = v_s[...] / jnp.maximum(s_s[...], 1e-16)
        wh = wsm_ref[:, 2:3]
        out_ref[...] = lax.dot_general(
            wh, gf, _AtB, preferred_element_type=jnp.float32) + scal_ref[0, 1]


def _readout(base, hid, segc, attc, st, w3, wsm, scal):
    return pl.pallas_call(
        _readout_body,
        grid=(NB,),
        in_specs=[
            pl.BlockSpec((EB, DH), lambda b: (b, 0)),
            pl.BlockSpec((EB, DH), lambda b: (b, 0)),
            pl.BlockSpec((1, EB, 1), lambda b: (b, 0, 0)),
            pl.BlockSpec((1, EB, 1), lambda b: (b, 0, 0)),
            pl.BlockSpec((G, 128), lambda b: (0, 0)),
            pl.BlockSpec((DH, DH), lambda b: (0, 0)),
            pl.BlockSpec((DH, 128), lambda b: (0, 0)),
            pl.BlockSpec(memory_space=pltpu.SMEM),
        ],
        out_specs=pl.BlockSpec((1, G), lambda b: (0, 0)),
        out_shape=jax.ShapeDtypeStruct((1, G), jnp.float32),
        scratch_shapes=[
            pltpu.VMEM((1, G), jnp.float32),
            pltpu.VMEM((1, G), jnp.float32),
            pltpu.VMEM((DH, G), jnp.float32),
        ],
        compiler_params=pltpu.CompilerParams(
            dimension_semantics=("arbitrary",)),
    )(base, hid, segc, attc, st, w3, wsm, scal)


def kernel(x, edge_index, edge_attr, batch,
           W_emb, b_emb, W_body, b_body, W_conf, b_conf,
           W_att, b_att, W_head, b_head):
    rol = edge_index[0]
    col = edge_index[1]

    wa = jnp.zeros((DF, 80), jnp.float32)
    wa = wa.at[:, 0:64].set(W_body[0:DF]).at[:, 64].set(W_att[0:DF, 0])
    wb = jnp.zeros((DF, 80), jnp.float32)
    wb = wb.at[:, 0:64].set(W_body[DF:2 * DF]).at[:, 64].set(
        W_att[DF:2 * DF, 0])
    wab = jnp.concatenate([wa, wb], axis=1)

    wep = jnp.zeros((DE, 192), jnp.float32)
    wep = (wep.at[:, 0:64].set(W_body[2 * DF + DH:])
           .at[:, 64:128].set(W_emb)
           .at[:, 128].set(W_att[2 * DF + DH:, 0]))
    bep = jnp.zeros((1, 192), jnp.float32)
    bep = (bep.at[0, 0:64].set(b_body)
           .at[0, 64:128].set(b_emb)
           .at[0, 128].set(b_att[0]))

    w3 = W_body[2 * DF:2 * DF + DH]
    wsm = jnp.zeros((DH, 128), jnp.float32)
    wsm = (wsm.at[:, 0].set(W_conf[:, 0])
           .at[:, 1].set(W_att[2 * DF:2 * DF + DH, 0])
           .at[:, 2].set(W_head[:, 0]))
    scal = jnp.stack([b_conf[0], b_head[0]]).reshape(1, 2)

    batchc = batch.reshape(N // NBLK, NBLK, 1)
    ya, yb, node_counts, brep = _node_proj(x, batchc, wab)
    yar, ybc, seg16 = _edge_gather(ya, yb, brep, rol, col)
    base, hid, attc, segc, st = _edge_combine(yar, ybc, edge_attr, seg16,
                                              wep, bep, w3, wsm, scal)
    attc = attc.reshape(NB, EB, 1)
    segc = segc.reshape(NB, EB, 1)
    for _ in range(ITERS - 1):
        hid, st = _iter_pass(base, hid, segc, st, w3, wsm, scal)
    out_row = _readout(base, hid, segc, attc, st, w3, wsm, scal)
    return (out_row.reshape(G, 1), node_counts)
